# Initial kernel scaffold; baseline (speedup 1.0000x reference)
import jax
import jax.numpy as jnp
from jax.experimental import pallas as pl


def kernel(x_drug, x_gene, edge_index_dd, edge_index_gd, edge_index_dg, w_dd, w_gd, w_dg, Wr_0_dd, br_0_dd, Wo_0_dd, Wr_0_gd, br_0_gd, Wo_0_gd, Wr_0_dg, br_0_dg, Wo_0_dg, Wr_1_dd, br_1_dd, Wo_1_dd, Wr_1_gd, br_1_gd, Wo_1_gd, Wr_1_dg, br_1_dg, Wo_1_dg, W1_mu_drug, b1_mu_drug, W2_mu_drug, b2_mu_drug, W1_ls_drug, b1_ls_drug, W2_ls_drug, b2_ls_drug, W1_mu_gene, b1_mu_gene, W2_mu_gene, b2_mu_gene, W1_ls_gene, b1_ls_gene, W2_ls_gene, b2_ls_gene):
    raise NotImplementedError("write your pallas kernel here")


# speedup vs baseline: 4.6005x; 4.6005x over previous
import functools

import jax
import jax.numpy as jnp
from jax import lax
from jax.experimental import pallas as pl
from jax.experimental.pallas import tpu as pltpu
from jax.experimental.pallas import tpu_sc as plsc

N = 50000
D_IN = 128
F0 = 64
F1 = 32
OUT = 32
NSUB = 16
NCORE = 2
NPAD = 50176
NROWS_P = NPAD // NSUB
ZCH = NROWS_P // 8
C_DD = 128
C_GD = 88
C_DG = 88
CMAX = 128
GRP = 4
R = 2000
GRID = N // R


def _pack_edges(ei, w, C):
    E = ei.shape[1]
    tot = NSUB * C * 128
    src = jnp.pad(ei[0], (0, tot - E)).reshape(NSUB, C, 128)
    dst = jnp.pad(ei[1], (0, tot - E)).reshape(NSUB, C, 128)
    wv = jnp.pad(w, (0, tot - E)).reshape(NSUB, C, 128)
    vv = jnp.pad(jnp.ones((E,), jnp.float32), (0, tot - E)).reshape(NSUB, C, 128)
    return src, dst, wv, vv


def _splat(v, e):
    idx = jnp.full((16, 1), e, dtype=jnp.int32)
    return lax.gather(
        v, idx,
        lax.GatherDimensionNumbers(
            offset_dims=(), collapsed_slice_dims=(0,), start_index_map=(0,)),
        (1,), mode=lax.GatherScatterMode.PROMISE_IN_BOUNDS)


def _make_prep():
    mesh = plsc.VectorSubcoreMesh(core_axis_name="c", subcore_axis_name="s")
    rel_cs = [C_DD, C_GD, C_DG]
    out_type = [jax.ShapeDtypeStruct((NSUB, C, 128), jnp.float32) for C in rel_cs]
    scratch = [
        pltpu.VMEM((NPAD,), jnp.float32),
        pltpu.VMEM((CMAX, 128), jnp.int32),
        pltpu.VMEM((CMAX, 128), jnp.float32),
        pltpu.VMEM((CMAX, 128), jnp.float32),
        pltpu.VMEM((CMAX, 128), jnp.float32),
        pltpu.VMEM_SHARED((NPAD,), jnp.float32),
    ]

    @functools.partial(pl.kernel, mesh=mesh, out_type=out_type,
                       scratch_types=scratch,
                       compiler_params=pltpu.CompilerParams(
                           needs_layout_passes=False,
                           use_tc_tiling_on_sc=False))
    def prep(z1, dst_dd, w_dd, v_dd, dst_gd, w_gd, v_gd, dst_dg, w_dg, v_dg,
             wq_dd, wq_gd, wq_dg, cbuf, dbuf, wbuf, vbuf, qbuf, cnt_sh):
        core = lax.axis_index("c")
        sid = lax.axis_index("s")
        ins = [(dst_dd, w_dd, v_dd, wq_dd, C_DD),
               (dst_gd, w_gd, v_gd, wq_gd, C_GD),
               (dst_dg, w_dg, v_dg, wq_dg, C_DG)]
        for dstr, wr, vr, wqr, C in ins:
            pltpu.sync_copy(z1, cbuf.at[pl.ds(0, NROWS_P)])
            pltpu.sync_copy(cbuf.at[pl.ds(0, NROWS_P)],
                            cnt_sh.at[pl.ds(sid * NROWS_P, NROWS_P)])
            plsc.subcore_barrier()
            pltpu.sync_copy(dstr.at[sid], dbuf.at[pl.ds(0, C)])
            pltpu.sync_copy(wr.at[sid], wbuf.at[pl.ds(0, C)])
            pltpu.sync_copy(vr.at[sid], vbuf.at[pl.ds(0, C)])

            def cnt_body(c, _):
                pltpu.sync_copy(vbuf.at[c], cnt_sh.at[dbuf.at[c]], add=True)
                return 0
            lax.fori_loop(0, C, cnt_body, 0)
            plsc.subcore_barrier()
            pltpu.sync_copy(cnt_sh, cbuf)

            def wq_body(c, _):
                for g in range(8):
                    sl = pl.ds(g * 16, 16)
                    c16 = plsc.load_gather(cbuf, [dbuf[c, sl]])
                    qbuf[c, sl] = wbuf[c, sl] / jnp.maximum(c16, 1.0)
                return 0
            lax.fori_loop(0, C, wq_body, 0)

            @pl.when(core == 0)
            def _():
                pltpu.sync_copy(qbuf.at[pl.ds(0, C)], wqr.at[sid])
    return prep


def _make_agg(Fc, rel_list):
    mesh = plsc.VectorSubcoreMesh(core_axis_name="c", subcore_axis_name="s")
    nrel = len(rel_list)
    out_type = jax.ShapeDtypeStruct((NCORE, NPAD, Fc), jnp.float32)
    scratch = [
        pltpu.VMEM((GRP, 128), jnp.int32),
        pltpu.VMEM((GRP, 128), jnp.int32),
        pltpu.VMEM((GRP, 128), jnp.float32),
        pltpu.VMEM((GRP * 128, Fc), jnp.float32),
        pltpu.VMEM_SHARED((NPAD, Fc), jnp.float32),
        pltpu.SemaphoreType.DMA,
    ]

    @functools.partial(pl.kernel, mesh=mesh, out_type=out_type,
                       scratch_types=scratch,
                       compiler_params=pltpu.CompilerParams(
                           needs_layout_passes=False,
                           use_tc_tiling_on_sc=False))
    def agg(Y, z2, *rest):
        edge_refs = rest[:3 * nrel]
        out = rest[3 * nrel]
        sbuf, dbuf, wbuf, rows, acc, sem = rest[3 * nrel + 1:]
        core = lax.axis_index("c")
        sid = lax.axis_index("s")
        base_rows = sid * NROWS_P
        pltpu.sync_copy(z2, rows.at[pl.ds(0, ZCH)])
        for k in range(8):
            pltpu.sync_copy(rows.at[pl.ds(0, ZCH)],
                            acc.at[pl.ds(base_rows + k * ZCH, ZCH)])
        plsc.subcore_barrier()
        for r, (rel_id, C) in enumerate(rel_list):
            srcr, dstr, wqr = edge_refs[3 * r:3 * r + 3]
            off = (core * 3 + rel_id) * N

            def grp_body(gi, _):
                b = gi * GRP
                pltpu.sync_copy(srcr.at[sid, pl.ds(b, GRP)], sbuf)
                pltpu.sync_copy(dstr.at[sid, pl.ds(b, GRP)], dbuf)
                pltpu.sync_copy(wqr.at[sid, pl.ds(b, GRP)], wbuf)

                def adj_body(j, _):
                    for g in range(8):
                        sl = pl.ds(g * 16, 16)
                        sbuf[j, sl] = sbuf[j, sl] + off
                    return 0
                lax.fori_loop(0, GRP, adj_body, 0)
                handles = [
                    pltpu.async_copy(Y.at[sbuf.at[j]],
                                     rows.at[pl.ds(j * 128, 128)], sem)
                    for j in range(GRP)]
                for h in handles:
                    h.wait()

                def mul_chunk(j, _):
                    def mul_g(g, _):
                        w16 = wbuf[j, pl.ds(g * 16, 16)]
                        for e in range(16):
                            sp = _splat(w16, e)
                            ro = j * 128 + g * 16 + e
                            for t in range(Fc // 16):
                                sl = pl.ds(t * 16, 16)
                                rows[ro, sl] = rows[ro, sl] * sp
                        return 0
                    lax.fori_loop(0, 8, mul_g, 0)
                    return 0
                lax.fori_loop(0, GRP, mul_chunk, 0)
                for j in range(GRP):
                    pltpu.sync_copy(rows.at[pl.ds(j * 128, 128)],
                                    acc.at[dbuf.at[j]], add=True)
                return 0
            lax.fori_loop(0, C // GRP, grp_body, 0)
        plsc.subcore_barrier()
        for k in range(8):
            lo = base_rows + k * ZCH
            pltpu.sync_copy(acc.at[pl.ds(lo, ZCH)], rows.at[pl.ds(0, ZCH)])
            pltpu.sync_copy(rows.at[pl.ds(0, ZCH)],
                            out.at[core, pl.ds(lo, ZCH)])
    return agg


def _norm_relu_rows(z):
    n = jnp.sqrt(jnp.sum(z * z, axis=1, keepdims=True))
    return jnp.maximum(z / jnp.maximum(n, 1e-12), 0.0)


def _dot(a, b):
    return jnp.dot(a, b, preferred_element_type=jnp.float32)


def _tc1_body(hd, hg, wr_dd, wr_gd, wr_dg, wo_dd, wo_gd, wo_dg,
              b_dd, b_gd, b_dg, y0, rootd, rootg):
    ydd = _dot(hd[...], wr_dd[...])
    ygd = _dot(hg[...], wr_gd[...])
    ydg = _dot(hd[...], wr_dg[...])
    y0[0] = ydd[:, :F0 // 2]
    y0[3] = ydd[:, F0 // 2:]
    y0[1] = ygd[:, :F0 // 2]
    y0[4] = ygd[:, F0 // 2:]
    y0[2] = ydg[:, :F0 // 2]
    y0[5] = ydg[:, F0 // 2:]
    rootd[...] = _dot(hd[...], wo_dd[...] + wo_gd[...]) + b_dd[...] + b_gd[...]
    rootg[...] = _dot(hg[...], wo_dg[...]) + b_dg[...]


def _tc1(hd, hg, wr_dd, wr_gd, wr_dg, wo_dd, wo_gd, wo_dg, b_dd, b_gd, b_dg):
    full = lambda shp: pl.BlockSpec(shp, lambda i: (0,) * len(shp))
    return pl.pallas_call(
        _tc1_body,
        grid=(GRID,),
        in_specs=[
            pl.BlockSpec((R, D_IN), lambda i: (i, 0)),
            pl.BlockSpec((R, D_IN), lambda i: (i, 0)),
            full((D_IN, F0)), full((D_IN, F0)), full((D_IN, F0)),
            full((D_IN, F0)), full((D_IN, F0)), full((D_IN, F0)),
            full((1, F0)), full((1, F0)), full((1, F0)),
        ],
        out_specs=[
            pl.BlockSpec((6, R, F0 // 2), lambda i: (0, i, 0)),
            pl.BlockSpec((R, F0), lambda i: (i, 0)),
            pl.BlockSpec((R, F0), lambda i: (i, 0)),
        ],
        out_shape=[
            jax.ShapeDtypeStruct((6, N, F0 // 2), jnp.float32),
            jax.ShapeDtypeStruct((N, F0), jnp.float32),
            jax.ShapeDtypeStruct((N, F0), jnp.float32),
        ],
    )(hd, hg, wr_dd, wr_gd, wr_dg, wo_dd, wo_gd, wo_dg, b_dd, b_gd, b_dg)


def _tc2_body(accd, rootd, accg, rootg, wr_dd, wr_gd, wr_dg,
              wo_dd, wo_gd, wo_dg, b_dd, b_gd, b_dg, y1, rootd1, rootg1):
    nd = jnp.concatenate([accd[0], accd[1]], axis=1) + rootd[...]
    ng = jnp.concatenate([accg[0], accg[1]], axis=1) + rootg[...]
    hd1 = _norm_relu_rows(nd)
    hg1 = _norm_relu_rows(ng)
    ydd = _dot(hd1, wr_dd[...])
    ygd = _dot(hg1, wr_gd[...])
    ydg = _dot(hd1, wr_dg[...])
    y1[0] = ydd[:, :F1 // 2]
    y1[3] = ydd[:, F1 // 2:]
    y1[1] = ygd[:, :F1 // 2]
    y1[4] = ygd[:, F1 // 2:]
    y1[2] = ydg[:, :F1 // 2]
    y1[5] = ydg[:, F1 // 2:]
    rootd1[...] = _dot(hd1, wo_dd[...] + wo_gd[...]) + b_dd[...] + b_gd[...]
    rootg1[...] = _dot(hg1, wo_dg[...]) + b_dg[...]


def _tc2(accd, rootd, accg, rootg, wr_dd, wr_gd, wr_dg,
         wo_dd, wo_gd, wo_dg, b_dd, b_gd, b_dg):
    full = lambda shp: pl.BlockSpec(shp, lambda i: (0,) * len(shp))
    return pl.pallas_call(
        _tc2_body,
        grid=(GRID,),
        in_specs=[
            pl.BlockSpec((2, R, F0 // 2), lambda i: (0, i, 0)),
            pl.BlockSpec((R, F0), lambda i: (i, 0)),
            pl.BlockSpec((2, R, F0 // 2), lambda i: (0, i, 0)),
            pl.BlockSpec((R, F0), lambda i: (i, 0)),
            full((F0, F1)), full((F0, F1)), full((F0, F1)),
            full((F0, F1)), full((F0, F1)), full((F0, F1)),
            full((1, F1)), full((1, F1)), full((1, F1)),
        ],
        out_specs=[
            pl.BlockSpec((6, R, F1 // 2), lambda i: (0, i, 0)),
            pl.BlockSpec((R, F1), lambda i: (i, 0)),
            pl.BlockSpec((R, F1), lambda i: (i, 0)),
        ],
        out_shape=[
            jax.ShapeDtypeStruct((6, N, F1 // 2), jnp.float32),
            jax.ShapeDtypeStruct((N, F1), jnp.float32),
            jax.ShapeDtypeStruct((N, F1), jnp.float32),
        ],
    )(accd, rootd, accg, rootg, wr_dd, wr_gd, wr_dg,
      wo_dd, wo_gd, wo_dg, b_dd, b_gd, b_dg)


def _head(h, w1mu, b1mu, w2mu, b2mu, w1ls, b1ls, w2ls, b2ls, noise):
    mu = _dot(jnp.maximum(_dot(h, w1mu[...]) + b1mu[...], 0.0), w2mu[...]) + b2mu[...]
    ls = _dot(jnp.maximum(_dot(h, w1ls[...]) + b1ls[...], 0.0), w2ls[...]) + b2ls[...]
    ls = jnp.minimum(ls, 10.0)
    return mu + noise[...] * jnp.exp(ls)


def _tc3_body(accd, rootd, accg, rootg,
              w1mud, b1mud, w2mud, b2mud, w1lsd, b1lsd, w2lsd, b2lsd,
              w1mug, b1mug, w2mug, b2mug, w1lsg, b1lsg, w2lsg, b2lsg,
              nd_, ng_, zd, zg):
    hd2 = _norm_relu_rows(jnp.concatenate([accd[0], accd[1]], axis=1) + rootd[...])
    hg2 = _norm_relu_rows(jnp.concatenate([accg[0], accg[1]], axis=1) + rootg[...])
    zd[...] = _head(hd2, w1mud, b1mud, w2mud, b2mud, w1lsd, b1lsd, w2lsd, b2lsd, nd_)
    zg[...] = _head(hg2, w1mug, b1mug, w2mug, b2mug, w1lsg, b1lsg, w2lsg, b2lsg, ng_)


def _tc3(accd, rootd, accg, rootg, hw, noise_d, noise_g):
    full = lambda shp: pl.BlockSpec(shp, lambda i: (0,) * len(shp))
    H2 = OUT // 2
    hw_specs = []
    for _ in range(2):
        for _ in range(2):
            hw_specs += [full((F1, H2)), full((1, H2)), full((H2, OUT)), full((1, OUT))]
    return pl.pallas_call(
        _tc3_body,
        grid=(GRID,),
        in_specs=[
            pl.BlockSpec((2, R, F1 // 2), lambda i: (0, i, 0)),
            pl.BlockSpec((R, F1), lambda i: (i, 0)),
            pl.BlockSpec((2, R, F1 // 2), lambda i: (0, i, 0)),
            pl.BlockSpec((R, F1), lambda i: (i, 0)),
            *hw_specs,
            pl.BlockSpec((R, OUT), lambda i: (i, 0)),
            pl.BlockSpec((R, OUT), lambda i: (i, 0)),
        ],
        out_specs=[
            pl.BlockSpec((R, OUT), lambda i: (i, 0)),
            pl.BlockSpec((R, OUT), lambda i: (i, 0)),
        ],
        out_shape=[
            jax.ShapeDtypeStruct((N, OUT), jnp.float32),
            jax.ShapeDtypeStruct((N, OUT), jnp.float32),
        ],
    )(accd, rootd, accg, rootg, *hw, noise_d, noise_g)


_prep_k = _make_prep()
_agg_d0 = _make_agg(F0 // 2, [(0, C_DD), (1, C_GD)])
_agg_g0 = _make_agg(F0 // 2, [(2, C_DG)])
_agg_d1 = _make_agg(F1 // 2, [(0, C_DD), (1, C_GD)])
_agg_g1 = _make_agg(F1 // 2, [(2, C_DG)])


def kernel(x_drug, x_gene, edge_index_dd, edge_index_gd, edge_index_dg,
           w_dd, w_gd, w_dg,
           Wr_0_dd, br_0_dd, Wo_0_dd, Wr_0_gd, br_0_gd, Wo_0_gd,
           Wr_0_dg, br_0_dg, Wo_0_dg,
           Wr_1_dd, br_1_dd, Wo_1_dd, Wr_1_gd, br_1_gd, Wo_1_gd,
           Wr_1_dg, br_1_dg, Wo_1_dg,
           W1_mu_drug, b1_mu_drug, W2_mu_drug, b2_mu_drug,
           W1_ls_drug, b1_ls_drug, W2_ls_drug, b2_ls_drug,
           W1_mu_gene, b1_mu_gene, W2_mu_gene, b2_mu_gene,
           W1_ls_gene, b1_ls_gene, W2_ls_gene, b2_ls_gene):
    f32 = jnp.float32
    src_dd, dst_dd, wv_dd, vv_dd = _pack_edges(edge_index_dd, w_dd, C_DD)
    src_gd, dst_gd, wv_gd, vv_gd = _pack_edges(edge_index_gd, w_gd, C_GD)
    src_dg, dst_dg, wv_dg, vv_dg = _pack_edges(edge_index_dg, w_dg, C_DG)
    z1 = jnp.zeros((NROWS_P,), f32)
    wq_dd, wq_gd, wq_dg = _prep_k(
        z1, dst_dd, wv_dd, vv_dd, dst_gd, wv_gd, vv_gd, dst_dg, wv_dg, vv_dg)

    r2 = lambda b: b.reshape(1, -1)
    y0, rootd0, rootg0 = _tc1(
        x_drug, x_gene, Wr_0_dd, Wr_0_gd, Wr_0_dg,
        Wo_0_dd, Wo_0_gd, Wo_0_dg, r2(br_0_dd), r2(br_0_gd), r2(br_0_dg))
    z2a = jnp.zeros((ZCH, F0 // 2), f32)
    accd0 = _agg_d0(y0.reshape(6 * N, F0 // 2), z2a,
                    src_dd, dst_dd, wq_dd, src_gd, dst_gd, wq_gd)
    accg0 = _agg_g0(y0.reshape(6 * N, F0 // 2), z2a, src_dg, dst_dg, wq_dg)

    y1, rootd1, rootg1 = _tc2(
        accd0, rootd0, accg0, rootg0, Wr_1_dd, Wr_1_gd, Wr_1_dg,
        Wo_1_dd, Wo_1_gd, Wo_1_dg, r2(br_1_dd), r2(br_1_gd), r2(br_1_dg))
    z2b = jnp.zeros((ZCH, F1 // 2), f32)
    accd1 = _agg_d1(y1.reshape(6 * N, F1 // 2), z2b,
                    src_dd, dst_dd, wq_dd, src_gd, dst_gd, wq_gd)
    accg1 = _agg_g1(y1.reshape(6 * N, F1 // 2), z2b, src_dg, dst_dg, wq_dg)

    noise_d = jax.random.normal(jax.random.key(42), (N, OUT), dtype=f32)
    noise_g = jax.random.normal(jax.random.key(43), (N, OUT), dtype=f32)
    hw = [W1_mu_drug, r2(b1_mu_drug), W2_mu_drug, r2(b2_mu_drug),
          W1_ls_drug, r2(b1_ls_drug), W2_ls_drug, r2(b2_ls_drug),
          W1_mu_gene, r2(b1_mu_gene), W2_mu_gene, r2(b2_mu_gene),
          W1_ls_gene, r2(b1_ls_gene), W2_ls_gene, r2(b2_ls_gene)]
    zd, zg = _tc3(accd1, rootd1, accg1, rootg1, hw, noise_d, noise_g)
    return jnp.concatenate([zd, zg], axis=0)


# speedup vs baseline: 5.0968x; 1.1079x over previous
import functools

import jax
import jax.numpy as jnp
from jax import lax
from jax.experimental import pallas as pl
from jax.experimental.pallas import tpu as pltpu
from jax.experimental.pallas import tpu_sc as plsc

N = 50000
D_IN = 128
F0 = 64
F1 = 32
OUT = 32
NSUB = 16
NCORE = 2
NPAD = 50176
NROWS_P = NPAD // NSUB
ZCH = 224
NZCH = NROWS_P // ZCH
C_DD = 126
C_GD = 90
C_DG = 90
CMAX = 126
GRP = 3
R = 2000
GRID = N // R


def _pack_edges(ei, w, C):
    E = ei.shape[1]
    tot = NSUB * C * 128
    src = jnp.pad(ei[0], (0, tot - E)).reshape(NSUB, C, 128)
    dst = jnp.pad(ei[1], (0, tot - E)).reshape(NSUB, C, 128)
    wv = jnp.pad(w, (0, tot - E)).reshape(NSUB, C, 128)
    vv = jnp.pad(jnp.ones((E,), jnp.float32), (0, tot - E)).reshape(NSUB, C, 128)
    return src, dst, wv, vv


def _splat(v, e):
    idx = jnp.full((16, 1), e, dtype=jnp.int32)
    return lax.gather(
        v, idx,
        lax.GatherDimensionNumbers(
            offset_dims=(), collapsed_slice_dims=(0,), start_index_map=(0,)),
        (1,), mode=lax.GatherScatterMode.PROMISE_IN_BOUNDS)


def _make_prep():
    mesh = plsc.VectorSubcoreMesh(core_axis_name="c", subcore_axis_name="s")
    rel_cs = [C_DD, C_GD, C_DG]
    out_type = [jax.ShapeDtypeStruct((NSUB, C, 128), jnp.float32) for C in rel_cs]
    scratch = [
        pltpu.VMEM((NPAD,), jnp.float32),
        pltpu.VMEM((CMAX, 128), jnp.int32),
        pltpu.VMEM((CMAX, 128), jnp.float32),
        pltpu.VMEM((CMAX, 128), jnp.float32),
        pltpu.VMEM((CMAX, 128), jnp.float32),
        pltpu.VMEM_SHARED((NPAD,), jnp.float32),
    ]

    @functools.partial(pl.kernel, mesh=mesh, out_type=out_type,
                       scratch_types=scratch,
                       compiler_params=pltpu.CompilerParams(
                           needs_layout_passes=False,
                           use_tc_tiling_on_sc=False))
    def prep(z1, dst_dd, w_dd, v_dd, dst_gd, w_gd, v_gd, dst_dg, w_dg, v_dg,
             wq_dd, wq_gd, wq_dg, cbuf, dbuf, wbuf, vbuf, qbuf, cnt_sh):
        core = lax.axis_index("c")
        sid = lax.axis_index("s")
        ins = [(dst_dd, w_dd, v_dd, wq_dd, C_DD),
               (dst_gd, w_gd, v_gd, wq_gd, C_GD),
               (dst_dg, w_dg, v_dg, wq_dg, C_DG)]
        for dstr, wr, vr, wqr, C in ins:
            pltpu.sync_copy(z1, cbuf.at[pl.ds(0, NROWS_P)])
            pltpu.sync_copy(cbuf.at[pl.ds(0, NROWS_P)],
                            cnt_sh.at[pl.ds(sid * NROWS_P, NROWS_P)])
            plsc.subcore_barrier()
            pltpu.sync_copy(dstr.at[sid], dbuf.at[pl.ds(0, C)])
            pltpu.sync_copy(wr.at[sid], wbuf.at[pl.ds(0, C)])
            pltpu.sync_copy(vr.at[sid], vbuf.at[pl.ds(0, C)])

            def cnt_body(c, _):
                pltpu.sync_copy(vbuf.at[c], cnt_sh.at[dbuf.at[c]], add=True)
                return 0
            lax.fori_loop(0, C, cnt_body, 0)
            plsc.subcore_barrier()
            pltpu.sync_copy(cnt_sh, cbuf)

            def wq_body(c, _):
                for g in range(8):
                    sl = pl.ds(g * 16, 16)
                    c16 = plsc.load_gather(cbuf, [dbuf[c, sl]])
                    qbuf[c, sl] = wbuf[c, sl] / jnp.maximum(c16, 1.0)
                return 0
            lax.fori_loop(0, C, wq_body, 0)

            @pl.when(core == 0)
            def _():
                pltpu.sync_copy(qbuf.at[pl.ds(0, C)], wqr.at[sid])
    return prep


def _make_agg(Fc, rel_list):
    mesh = plsc.VectorSubcoreMesh(core_axis_name="c", subcore_axis_name="s")
    nrel = len(rel_list)
    out_type = jax.ShapeDtypeStruct((NCORE, NPAD, Fc), jnp.float32)
    scratch = [
        pltpu.VMEM((GRP, 128), jnp.int32),
        pltpu.VMEM((GRP, 128), jnp.int32),
        pltpu.VMEM((GRP, 128), jnp.int32),
        pltpu.VMEM((GRP, 128), jnp.int32),
        pltpu.VMEM((GRP, 128), jnp.float32),
        pltpu.VMEM((GRP, 128), jnp.float32),
        pltpu.VMEM((GRP * 128, Fc), jnp.float32),
        pltpu.VMEM((GRP * 128, Fc), jnp.float32),
        pltpu.VMEM_SHARED((NPAD, Fc), jnp.float32),
        pltpu.SemaphoreType.DMA,
        pltpu.SemaphoreType.DMA,
        pltpu.SemaphoreType.DMA,
        pltpu.SemaphoreType.DMA,
        pltpu.SemaphoreType.DMA,
    ]

    @functools.partial(pl.kernel, mesh=mesh, out_type=out_type,
                       scratch_types=scratch,
                       compiler_params=pltpu.CompilerParams(
                           needs_layout_passes=False,
                           use_tc_tiling_on_sc=False))
    def agg(Y, z2, *rest):
        edge_refs = rest[:3 * nrel]
        out = rest[3 * nrel]
        (sbuf0, sbuf1, dbuf0, dbuf1, wbuf0, wbuf1, rows0, rows1, acc,
         semi, semg0, semg1, sems0, sems1) = rest[3 * nrel + 1:]
        sbuf = [sbuf0, sbuf1]
        dbuf = [dbuf0, dbuf1]
        wbuf = [wbuf0, wbuf1]
        rows = [rows0, rows1]
        semg = [semg0, semg1]
        sems = [sems0, sems1]
        core = lax.axis_index("c")
        sid = lax.axis_index("s")
        base_rows = sid * NROWS_P
        pltpu.sync_copy(z2, rows0.at[pl.ds(0, ZCH)])
        for k in range(NZCH):
            pltpu.sync_copy(rows0.at[pl.ds(0, ZCH)],
                            acc.at[pl.ds(base_rows + k * ZCH, ZCH)])
        plsc.subcore_barrier()

        def mul(p):
            def mul_chunk(j, _):
                def mul_g(g, _):
                    w16 = wbuf[p][j, pl.ds(g * 16, 16)]
                    for e in range(16):
                        sp = _splat(w16, e)
                        ro = j * 128 + g * 16 + e
                        for t in range(Fc // 16):
                            sl = pl.ds(t * 16, 16)
                            rows[p][ro, sl] = rows[p][ro, sl] * sp
                    return 0
                lax.fori_loop(0, 8, mul_g, 0)
                return 0
            lax.fori_loop(0, GRP, mul_chunk, 0)

        for r, (rel_id, C) in enumerate(rel_list):
            srcr, dstr, wqr = edge_refs[3 * r:3 * r + 3]
            off = (core * 3 + rel_id) * N
            NG = C // GRP

            def load_adj_fire(p, g):
                b = g * GRP
                h = [pltpu.async_copy(srcr.at[sid, pl.ds(b, GRP)], sbuf[p], semi),
                     pltpu.async_copy(dstr.at[sid, pl.ds(b, GRP)], dbuf[p], semi),
                     pltpu.async_copy(wqr.at[sid, pl.ds(b, GRP)], wbuf[p], semi)]
                for hh in h:
                    hh.wait()

                def adj_body(j, _):
                    for gg in range(8):
                        sl = pl.ds(gg * 16, 16)
                        sbuf[p][j, sl] = sbuf[p][j, sl] + off
                    return 0
                lax.fori_loop(0, GRP, adj_body, 0)
                for j in range(GRP):
                    pltpu.async_copy(Y.at[sbuf[p].at[j]],
                                     rows[p].at[pl.ds(j * 128, 128)], semg[p])

            def wait_gathers(p):
                for j in range(GRP):
                    pltpu.make_async_copy(
                        Y.at[sbuf[p].at[j]],
                        rows[p].at[pl.ds(j * 128, 128)], semg[p]).wait()

            def fire_scatters(p):
                for j in range(GRP):
                    pltpu.async_copy(rows[p].at[pl.ds(j * 128, 128)],
                                     acc.at[dbuf[p].at[j]], sems[p], add=True)

            def wait_scatters(p):
                for j in range(GRP):
                    pltpu.make_async_copy(
                        rows[p].at[pl.ds(j * 128, 128)],
                        acc.at[dbuf[p].at[j]], sems[p]).wait()

            load_adj_fire(0, 0)

            def pair_body(k, _):
                wait_gathers(0)

                @pl.when(k > 0)
                def _():
                    wait_scatters(1)
                load_adj_fire(1, 2 * k + 1)
                mul(0)
                fire_scatters(0)
                wait_gathers(1)

                @pl.when(k < NG // 2 - 1)
                def _():
                    wait_scatters(0)
                    load_adj_fire(0, 2 * k + 2)
                mul(1)
                fire_scatters(1)
                return 0
            lax.fori_loop(0, NG // 2, pair_body, 0)
            wait_scatters(0)
            wait_scatters(1)
        plsc.subcore_barrier()
        for k in range(NZCH):
            lo = base_rows + k * ZCH
            pltpu.sync_copy(acc.at[pl.ds(lo, ZCH)], rows0.at[pl.ds(0, ZCH)])
            pltpu.sync_copy(rows0.at[pl.ds(0, ZCH)],
                            out.at[core, pl.ds(lo, ZCH)])
    return agg


def _norm_relu_rows(z):
    n = jnp.sqrt(jnp.sum(z * z, axis=1, keepdims=True))
    return jnp.maximum(z / jnp.maximum(n, 1e-12), 0.0)


def _dot(a, b):
    return jnp.dot(a, b, preferred_element_type=jnp.float32)


def _tc1_body(hd, hg, wr_dd, wr_gd, wr_dg, wo_dd, wo_gd, wo_dg,
              b_dd, b_gd, b_dg, y0, rootd, rootg):
    ydd = _dot(hd[...], wr_dd[...])
    ygd = _dot(hg[...], wr_gd[...])
    ydg = _dot(hd[...], wr_dg[...])
    y0[0] = ydd[:, :F0 // 2]
    y0[3] = ydd[:, F0 // 2:]
    y0[1] = ygd[:, :F0 // 2]
    y0[4] = ygd[:, F0 // 2:]
    y0[2] = ydg[:, :F0 // 2]
    y0[5] = ydg[:, F0 // 2:]
    rootd[...] = _dot(hd[...], wo_dd[...] + wo_gd[...]) + b_dd[...] + b_gd[...]
    rootg[...] = _dot(hg[...], wo_dg[...]) + b_dg[...]


def _tc1(hd, hg, wr_dd, wr_gd, wr_dg, wo_dd, wo_gd, wo_dg, b_dd, b_gd, b_dg):
    full = lambda shp: pl.BlockSpec(shp, lambda i: (0,) * len(shp))
    return pl.pallas_call(
        _tc1_body,
        grid=(GRID,),
        in_specs=[
            pl.BlockSpec((R, D_IN), lambda i: (i, 0)),
            pl.BlockSpec((R, D_IN), lambda i: (i, 0)),
            full((D_IN, F0)), full((D_IN, F0)), full((D_IN, F0)),
            full((D_IN, F0)), full((D_IN, F0)), full((D_IN, F0)),
            full((1, F0)), full((1, F0)), full((1, F0)),
        ],
        out_specs=[
            pl.BlockSpec((6, R, F0 // 2), lambda i: (0, i, 0)),
            pl.BlockSpec((R, F0), lambda i: (i, 0)),
            pl.BlockSpec((R, F0), lambda i: (i, 0)),
        ],
        out_shape=[
            jax.ShapeDtypeStruct((6, N, F0 // 2), jnp.float32),
            jax.ShapeDtypeStruct((N, F0), jnp.float32),
            jax.ShapeDtypeStruct((N, F0), jnp.float32),
        ],
    )(hd, hg, wr_dd, wr_gd, wr_dg, wo_dd, wo_gd, wo_dg, b_dd, b_gd, b_dg)


def _tc2_body(accd, rootd, accg, rootg, wr_dd, wr_gd, wr_dg,
              wo_dd, wo_gd, wo_dg, b_dd, b_gd, b_dg, y1, rootd1, rootg1):
    nd = jnp.concatenate([accd[0], accd[1]], axis=1) + rootd[...]
    ng = jnp.concatenate([accg[0], accg[1]], axis=1) + rootg[...]
    hd1 = _norm_relu_rows(nd)
    hg1 = _norm_relu_rows(ng)
    ydd = _dot(hd1, wr_dd[...])
    ygd = _dot(hg1, wr_gd[...])
    ydg = _dot(hd1, wr_dg[...])
    y1[0] = ydd[:, :F1 // 2]
    y1[3] = ydd[:, F1 // 2:]
    y1[1] = ygd[:, :F1 // 2]
    y1[4] = ygd[:, F1 // 2:]
    y1[2] = ydg[:, :F1 // 2]
    y1[5] = ydg[:, F1 // 2:]
    rootd1[...] = _dot(hd1, wo_dd[...] + wo_gd[...]) + b_dd[...] + b_gd[...]
    rootg1[...] = _dot(hg1, wo_dg[...]) + b_dg[...]


def _tc2(accd, rootd, accg, rootg, wr_dd, wr_gd, wr_dg,
         wo_dd, wo_gd, wo_dg, b_dd, b_gd, b_dg):
    full = lambda shp: pl.BlockSpec(shp, lambda i: (0,) * len(shp))
    return pl.pallas_call(
        _tc2_body,
        grid=(GRID,),
        in_specs=[
            pl.BlockSpec((2, R, F0 // 2), lambda i: (0, i, 0)),
            pl.BlockSpec((R, F0), lambda i: (i, 0)),
            pl.BlockSpec((2, R, F0 // 2), lambda i: (0, i, 0)),
            pl.BlockSpec((R, F0), lambda i: (i, 0)),
            full((F0, F1)), full((F0, F1)), full((F0, F1)),
            full((F0, F1)), full((F0, F1)), full((F0, F1)),
            full((1, F1)), full((1, F1)), full((1, F1)),
        ],
        out_specs=[
            pl.BlockSpec((6, R, F1 // 2), lambda i: (0, i, 0)),
            pl.BlockSpec((R, F1), lambda i: (i, 0)),
            pl.BlockSpec((R, F1), lambda i: (i, 0)),
        ],
        out_shape=[
            jax.ShapeDtypeStruct((6, N, F1 // 2), jnp.float32),
            jax.ShapeDtypeStruct((N, F1), jnp.float32),
            jax.ShapeDtypeStruct((N, F1), jnp.float32),
        ],
    )(accd, rootd, accg, rootg, wr_dd, wr_gd, wr_dg,
      wo_dd, wo_gd, wo_dg, b_dd, b_gd, b_dg)


def _head(h, w1mu, b1mu, w2mu, b2mu, w1ls, b1ls, w2ls, b2ls, noise):
    mu = _dot(jnp.maximum(_dot(h, w1mu[...]) + b1mu[...], 0.0), w2mu[...]) + b2mu[...]
    ls = _dot(jnp.maximum(_dot(h, w1ls[...]) + b1ls[...], 0.0), w2ls[...]) + b2ls[...]
    ls = jnp.minimum(ls, 10.0)
    return mu + noise[...] * jnp.exp(ls)


def _tc3_body(accd, rootd, accg, rootg,
              w1mud, b1mud, w2mud, b2mud, w1lsd, b1lsd, w2lsd, b2lsd,
              w1mug, b1mug, w2mug, b2mug, w1lsg, b1lsg, w2lsg, b2lsg,
              nd_, ng_, zd, zg):
    hd2 = _norm_relu_rows(jnp.concatenate([accd[0], accd[1]], axis=1) + rootd[...])
    hg2 = _norm_relu_rows(jnp.concatenate([accg[0], accg[1]], axis=1) + rootg[...])
    zd[...] = _head(hd2, w1mud, b1mud, w2mud, b2mud, w1lsd, b1lsd, w2lsd, b2lsd, nd_)
    zg[...] = _head(hg2, w1mug, b1mug, w2mug, b2mug, w1lsg, b1lsg, w2lsg, b2lsg, ng_)


def _tc3(accd, rootd, accg, rootg, hw, noise_d, noise_g):
    full = lambda shp: pl.BlockSpec(shp, lambda i: (0,) * len(shp))
    H2 = OUT // 2
    hw_specs = []
    for _ in range(2):
        for _ in range(2):
            hw_specs += [full((F1, H2)), full((1, H2)), full((H2, OUT)), full((1, OUT))]
    return pl.pallas_call(
        _tc3_body,
        grid=(GRID,),
        in_specs=[
            pl.BlockSpec((2, R, F1 // 2), lambda i: (0, i, 0)),
            pl.BlockSpec((R, F1), lambda i: (i, 0)),
            pl.BlockSpec((2, R, F1 // 2), lambda i: (0, i, 0)),
            pl.BlockSpec((R, F1), lambda i: (i, 0)),
            *hw_specs,
            pl.BlockSpec((R, OUT), lambda i: (i, 0)),
            pl.BlockSpec((R, OUT), lambda i: (i, 0)),
        ],
        out_specs=[
            pl.BlockSpec((R, OUT), lambda i: (i, 0)),
            pl.BlockSpec((R, OUT), lambda i: (i, 0)),
        ],
        out_shape=[
            jax.ShapeDtypeStruct((N, OUT), jnp.float32),
            jax.ShapeDtypeStruct((N, OUT), jnp.float32),
        ],
    )(accd, rootd, accg, rootg, *hw, noise_d, noise_g)


_prep_k = _make_prep()
_agg_d0 = _make_agg(F0 // 2, [(0, C_DD), (1, C_GD)])
_agg_g0 = _make_agg(F0 // 2, [(2, C_DG)])
_agg_d1 = _make_agg(F1 // 2, [(0, C_DD), (1, C_GD)])
_agg_g1 = _make_agg(F1 // 2, [(2, C_DG)])


def kernel(x_drug, x_gene, edge_index_dd, edge_index_gd, edge_index_dg,
           w_dd, w_gd, w_dg,
           Wr_0_dd, br_0_dd, Wo_0_dd, Wr_0_gd, br_0_gd, Wo_0_gd,
           Wr_0_dg, br_0_dg, Wo_0_dg,
           Wr_1_dd, br_1_dd, Wo_1_dd, Wr_1_gd, br_1_gd, Wo_1_gd,
           Wr_1_dg, br_1_dg, Wo_1_dg,
           W1_mu_drug, b1_mu_drug, W2_mu_drug, b2_mu_drug,
           W1_ls_drug, b1_ls_drug, W2_ls_drug, b2_ls_drug,
           W1_mu_gene, b1_mu_gene, W2_mu_gene, b2_mu_gene,
           W1_ls_gene, b1_ls_gene, W2_ls_gene, b2_ls_gene):
    f32 = jnp.float32
    src_dd, dst_dd, wv_dd, vv_dd = _pack_edges(edge_index_dd, w_dd, C_DD)
    src_gd, dst_gd, wv_gd, vv_gd = _pack_edges(edge_index_gd, w_gd, C_GD)
    src_dg, dst_dg, wv_dg, vv_dg = _pack_edges(edge_index_dg, w_dg, C_DG)
    z1 = jnp.zeros((NROWS_P,), f32)
    wq_dd, wq_gd, wq_dg = _prep_k(
        z1, dst_dd, wv_dd, vv_dd, dst_gd, wv_gd, vv_gd, dst_dg, wv_dg, vv_dg)

    r2 = lambda b: b.reshape(1, -1)
    y0, rootd0, rootg0 = _tc1(
        x_drug, x_gene, Wr_0_dd, Wr_0_gd, Wr_0_dg,
        Wo_0_dd, Wo_0_gd, Wo_0_dg, r2(br_0_dd), r2(br_0_gd), r2(br_0_dg))
    z2a = jnp.zeros((ZCH, F0 // 2), f32)
    accd0 = _agg_d0(y0.reshape(6 * N, F0 // 2), z2a,
                    src_dd, dst_dd, wq_dd, src_gd, dst_gd, wq_gd)
    accg0 = _agg_g0(y0.reshape(6 * N, F0 // 2), z2a, src_dg, dst_dg, wq_dg)

    y1, rootd1, rootg1 = _tc2(
        accd0, rootd0, accg0, rootg0, Wr_1_dd, Wr_1_gd, Wr_1_dg,
        Wo_1_dd, Wo_1_gd, Wo_1_dg, r2(br_1_dd), r2(br_1_gd), r2(br_1_dg))
    z2b = jnp.zeros((ZCH, F1 // 2), f32)
    accd1 = _agg_d1(y1.reshape(6 * N, F1 // 2), z2b,
                    src_dd, dst_dd, wq_dd, src_gd, dst_gd, wq_gd)
    accg1 = _agg_g1(y1.reshape(6 * N, F1 // 2), z2b, src_dg, dst_dg, wq_dg)

    noise_d = jax.random.normal(jax.random.key(42), (N, OUT), dtype=f32)
    noise_g = jax.random.normal(jax.random.key(43), (N, OUT), dtype=f32)
    hw = [W1_mu_drug, r2(b1_mu_drug), W2_mu_drug, r2(b2_mu_drug),
          W1_ls_drug, r2(b1_ls_drug), W2_ls_drug, r2(b2_ls_drug),
          W1_mu_gene, r2(b1_mu_gene), W2_mu_gene, r2(b2_mu_gene),
          W1_ls_gene, r2(b1_ls_gene), W2_ls_gene, r2(b2_ls_gene)]
    zd, zg = _tc3(accd1, rootd1, accg1, rootg1, hw, noise_d, noise_g)
    return jnp.concatenate([zd, zg], axis=0)
